# baseline (speedup 1.0000x reference)
import functools

import jax
import jax.numpy as jnp
from jax import lax
from jax.experimental import pallas as pl
from jax.experimental.pallas import tpu as pltpu
from jax.experimental.pallas import tpu_sc as plsc

B = 4
V = 100000
F = 200000
W = 16
NC = 2
NS = 16
NW = NC * NS
K = 128
CHUNKS = 100
FT = K * CHUNKS
F_PAD = NS * FT
F_ALLOC = F_PAD + 2 * K
V_PAD = 100352
RPS = V_PAD // NS
PK = 64
PIECES = RPS // PK
GPC = K // 16

_mesh = plsc.VectorSubcoreMesh(core_axis_name="c", subcore_axis_name="s")


def _sc_scratch():
    s = [pltpu.VMEM_SHARED((V_PAD, W), jnp.float32)]
    s += [pltpu.VMEM((PK,), jnp.float32) for _ in range(12)]
    s += [pltpu.VMEM((PK, W), jnp.float32) for _ in range(2)]
    s += [pltpu.VMEM((K, W), jnp.float32) for _ in range(6)]
    s += [pltpu.VMEM((K, W), jnp.float32) for _ in range(2)]
    s += [pltpu.VMEM((K,), jnp.int32) for _ in range(18)]
    s += [pltpu.VMEM((PK, W), jnp.float32) for _ in range(2)]
    s += [pltpu.VMEM((2, 3, PK), jnp.float32) for _ in range(2)]
    s += [pltpu.SemaphoreType.DMA for _ in range(6)]
    return s


@functools.partial(
    pl.kernel,
    out_type=(jax.ShapeDtypeStruct((NC, 2, 3, V_PAD), jnp.float32),
              jax.ShapeDtypeStruct((NC * V_PAD, W), jnp.float32)),
    mesh=_mesh,
    scratch_types=_sc_scratch(),
    compiler_params=pltpu.CompilerParams(use_tc_tiling_on_sc=False,
                                         needs_layout_passes=False),
)
def _vertex_normals_sc(v12, f3, o_hbm, pack, *rest):
    acc = rest[0]
    pc = rest[1:13]
    pp = rest[13:15]
    gb = rest[15:21]
    nb = rest[21:23]
    ix = rest[23:29]
    gix = rest[29:35]
    sidx = rest[35:41]
    ab = rest[41:43]
    ob = rest[43:45]
    sem_a, sem_b, sem_s0, sem_s1, sem_f0, sem_f1 = rest[45:51]
    sem_g = (sem_a, sem_b)
    sem_s = (sem_s0, sem_s1)
    sem_f = (sem_f0, sem_f1)

    cid = lax.axis_index("c")
    sid = lax.axis_index("s")
    tid = cid * NS + sid
    row0 = sid * RPS
    fbase = sid * FT
    vbase = cid * V_PAD
    lanes = lax.iota(jnp.int32, 16)
    zrow = jnp.zeros((W,), jnp.float32)

    def _zrows(k, carry):
        nb[0][k] = zrow
        nb[1][k] = zrow
        return carry
    lax.fori_loop(0, K, _zrows, 0)

    def _zpp(k, carry):
        pp[0][k] = zrow
        pp[1][k] = zrow
        return carry
    lax.fori_loop(0, PK, _zpp, 0)

    def _zacc(i, carry):
        pltpu.sync_copy(nb[0], acc.at[pl.ds(row0 + i * K, K)])
        return carry
    lax.fori_loop(0, RPS // K, _zacc, 0)

    def _fire_pc(slot, p):
        ps = jnp.minimum(p, PIECES - 1)
        r = jnp.minimum(row0 + ps * PK, V - PK)
        for q in range(2):
            for c in range(3):
                pr = c * 4 + 2 * cid + q
                pltpu.async_copy(v12.at[pr, pl.ds(r, PK)],
                                 pc[slot * 6 + q * 3 + c], sem_g[slot])

    def _wait_pc(slot):
        for i in range(6):
            pltpu.make_async_copy(v12.at[0, pl.ds(0, PK)],
                                  pc[slot * 6 + i], sem_g[slot]).wait()

    _fire_pc(0, 0)

    def _pack_pair(t, carry):
        for s in range(2):
            p = 2 * t + s
            _fire_pc(1 - s, p + 1)
            _wait_pc(s)

            @pl.when(t >= 1)
            def _():
                pltpu.make_async_copy(pp[s], pack.at[pl.ds(0, PK)],
                                      sem_s[s]).wait()

            for gg in range(PK // 16):
                rows = gg * 16 + lanes
                for lp in range(6):
                    vals = pc[s * 6 + lp][pl.ds(gg * 16, 16)]
                    plsc.store_scatter(pp[s],
                                       [rows, jnp.full((16,), lp, jnp.int32)],
                                       vals)
            rc = jnp.minimum(row0 + p * PK, V - PK)
            pltpu.async_copy(pp[s], pack.at[pl.ds(vbase + rc, PK)],
                             sem_s[s])
        return carry
    lax.fori_loop(0, PIECES // 2, _pack_pair, 0)

    for s in range(2):
        pltpu.make_async_copy(pp[s], pack.at[pl.ds(0, PK)], sem_s[s]).wait()
    _wait_pc(0)

    plsc.subcore_barrier()

    def _fire_ix(slot, j):
        for c in range(3):
            pltpu.async_copy(f3.at[c, pl.ds(fbase + j * K, K)],
                             ix[slot * 3 + c], sem_f[slot])

    def _wait_ix(slot):
        for c in range(3):
            pltpu.make_async_copy(f3.at[0, pl.ds(0, K)],
                                  ix[slot * 3 + c], sem_f[slot]).wait()

    def _mk_gix(slot):
        for c in range(3):
            for gg in range(GPC):
                sl = pl.ds(gg * 16, 16)
                gix[slot * 3 + c][sl] = ix[slot * 3 + c][sl] + vbase

    def _fire_gathers(slot):
        for c in range(3):
            pltpu.async_copy(pack.at[gix[slot * 3 + c]], gb[slot * 3 + c],
                             sem_g[slot])

    def _wait_gathers(slot):
        for c in range(3):
            pltpu.make_async_copy(pack.at[gix[slot * 3 + c]],
                                  gb[slot * 3 + c], sem_g[slot]).wait()

    def _fire_scatters(slot):
        for c in range(3):
            pltpu.async_copy(nb[slot], acc.at[sidx[slot * 3 + c]],
                             sem_s[slot], add=True)

    def _wait_scatters(slot):
        for c in range(3):
            pltpu.make_async_copy(nb[slot], acc.at[sidx[slot * 3 + c]],
                                  sem_s[slot]).wait()

    def _copy_sidx(slot):
        for c in range(3):
            for gg in range(GPC):
                sl = pl.ds(gg * 16, 16)
                sidx[slot * 3 + c][sl] = ix[slot * 3 + c][sl]

    def _compute(slot):
        for gg in range(GPC):
            rows = gg * 16 + lanes
            for q in range(2):
                cols = [jnp.full((16,), q * 3 + cc, jnp.int32)
                        for cc in range(3)]
                v0 = [plsc.load_gather(gb[slot * 3 + 0], [rows, cols[cc]])
                      for cc in range(3)]
                v1 = [plsc.load_gather(gb[slot * 3 + 1], [rows, cols[cc]])
                      for cc in range(3)]
                v2 = [plsc.load_gather(gb[slot * 3 + 2], [rows, cols[cc]])
                      for cc in range(3)]
                e1 = [v0[cc] - v1[cc] for cc in range(3)]
                e2 = [v2[cc] - v1[cc] for cc in range(3)]
                for cc in range(3):
                    n = (e2[(cc + 1) % 3] * e1[(cc + 2) % 3]
                         - e2[(cc + 2) % 3] * e1[(cc + 1) % 3])
                    plsc.store_scatter(nb[slot], [rows, cols[cc]], n)

    _fire_ix(0, 0)
    _wait_ix(0)
    _mk_gix(0)
    _fire_gathers(0)
    _fire_ix(1, 1)

    def _pair(t, carry):
        for s in range(2):
            j = 2 * t + s
            _wait_ix(1 - s)
            _mk_gix(1 - s)
            _fire_gathers(1 - s)
            _wait_gathers(s)

            @pl.when(t >= 1)
            def _():
                _wait_scatters(s)

            _compute(s)
            _copy_sidx(s)
            _fire_ix(s, j + 2)
            _fire_scatters(s)
        return carry
    lax.fori_loop(0, CHUNKS // 2, _pair, 0)

    _wait_gathers(0)
    _wait_ix(1)
    _wait_scatters(0)
    _wait_scatters(1)

    plsc.subcore_barrier()

    def _rsqrt(x):
        xi = plsc.bitcast(x, jnp.int32)
        yi = jnp.full((16,), 0x5F3759DF, jnp.int32) - \
            lax.shift_right_logical(xi, 1)
        y = plsc.bitcast(yi, jnp.float32)
        h = x * 0.5
        for _ in range(3):
            y = y * (1.5 - h * y * y)
        return y

    def _fire_ab(slot, p):
        ps = jnp.minimum(p, PIECES - 1)
        pltpu.async_copy(acc.at[pl.ds(row0 + ps * PK, PK)], ab[slot],
                         sem_g[slot])

    def _wait_ab(slot):
        pltpu.make_async_copy(acc.at[pl.ds(0, PK)], ab[slot],
                              sem_g[slot]).wait()

    _fire_ab(0, 0)

    def _norm_pair(t, carry):
        for s in range(2):
            p = 2 * t + s
            _fire_ab(1 - s, p + 1)
            _wait_ab(s)

            @pl.when(t >= 1)
            def _():
                pltpu.make_async_copy(ob[s], o_hbm.at[0, :, :, pl.ds(0, PK)],
                                      sem_s[s]).wait()

            for gg in range(PK // 16):
                rows = gg * 16 + lanes
                for q in range(2):
                    x = plsc.load_gather(
                        ab[s], [rows, jnp.full((16,), q * 3, jnp.int32)])
                    y = plsc.load_gather(
                        ab[s], [rows, jnp.full((16,), q * 3 + 1, jnp.int32)])
                    z = plsc.load_gather(
                        ab[s], [rows, jnp.full((16,), q * 3 + 2, jnp.int32)])
                    ss = jnp.maximum(x * x + y * y + z * z,
                                     jnp.full((16,), 1e-12, jnp.float32))
                    r = _rsqrt(ss)
                    ob[s][q, 0, pl.ds(gg * 16, 16)] = x * r
                    ob[s][q, 1, pl.ds(gg * 16, 16)] = y * r
                    ob[s][q, 2, pl.ds(gg * 16, 16)] = z * r
            pltpu.async_copy(ob[s],
                             o_hbm.at[cid, :, :, pl.ds(row0 + p * PK, PK)],
                             sem_s[s])
        return carry
    lax.fori_loop(0, PIECES // 2, _norm_pair, 0)

    for s in range(2):
        pltpu.make_async_copy(ob[s], o_hbm.at[0, :, :, pl.ds(0, PK)],
                              sem_s[s]).wait()
    _wait_ab(0)


def kernel(vertices, faces):
    v12 = jnp.transpose(vertices, (2, 0, 1)).reshape(B * 3, V)
    f3 = jnp.zeros((3, F_ALLOC), jnp.int32).at[:, :F].set(faces.T)
    o, _ = _vertex_normals_sc(v12, f3)
    o = o.reshape(B, 3, V_PAD)[:, :, :V]
    return jnp.transpose(o, (0, 2, 1))
